# Initial kernel scaffold; baseline (speedup 1.0000x reference)
import jax
import jax.numpy as jnp
from jax.experimental import pallas as pl


def kernel(ram):
    raise NotImplementedError("write your pallas kernel here")


# speedup vs baseline: 14.0899x; 14.0899x over previous
import functools

import jax
import jax.numpy as jnp
from jax import lax
from jax.experimental import pallas as pl
from jax.experimental.pallas import tpu as pltpu
from jax.experimental.pallas import tpu_sc as plsc

N_ROWS = 262144
N_SUBCORES = 16
ROWS_PER = N_ROWS // N_SUBCORES
N_LANES = 16
N_STAGES = 2
STAGE_ROWS = ROWS_PER // N_STAGES
GROUPS = STAGE_ROWS // N_LANES
COL0 = 32


def _sc_body(ram_hbm, out_hbm, buf, ind_v, idx_v, z_v, acc_v, out_v, shared):
    wid = lax.axis_index("s")
    lanes = lax.iota(jnp.int32, N_LANES)

    def body(i, bits):
        rows = i * N_LANES + lanes
        c0 = jnp.zeros((N_LANES,), jnp.int32)
        mi_x = plsc.load_gather(buf, [rows, c0])
        su_x = plsc.load_gather(buf, [rows, c0 + 1])
        mi_y = plsc.load_gather(buf, [rows, c0 + 2])
        su_y = plsc.load_gather(buf, [rows, c0 + 3])

        dist_x = jnp.abs(su_x - mi_x)
        dist_y = jnp.abs(su_y - mi_y)
        go_down = su_y > mi_y
        go_up = su_y < mi_y
        go_right = su_x > mi_x
        go_left = su_x < mi_x
        cond_y = dist_y > 2.0
        cond_x_far = dist_x > 24.0
        cond_x_close = dist_x < 22.0

        i2 = jnp.full((N_LANES,), 2, jnp.int32)
        act_y = jnp.where(go_down, jnp.full((N_LANES,), 5, jnp.int32), i2)
        act_x_far = jnp.where(
            go_right, jnp.full((N_LANES,), 3, jnp.int32),
            jnp.full((N_LANES,), 4, jnp.int32))
        act_x_close = jnp.where(
            go_right, jnp.full((N_LANES,), 4, jnp.int32),
            jnp.full((N_LANES,), 3, jnp.int32))

        up_left = go_up & go_left
        down_right = go_down & go_right
        down_left = go_down & go_left

        adf = jnp.full((N_LANES,), 6, jnp.int32)
        adf = jnp.where(up_left, jnp.full((N_LANES,), 7, jnp.int32), adf)
        adf = jnp.where(down_right, jnp.full((N_LANES,), 8, jnp.int32), adf)
        adf = jnp.where(down_left, jnp.full((N_LANES,), 9, jnp.int32), adf)

        adc = jnp.full((N_LANES,), 7, jnp.int32)
        adc = jnp.where(up_left, jnp.full((N_LANES,), 6, jnp.int32), adc)
        adc = jnp.where(down_right, jnp.full((N_LANES,), 9, jnp.int32), adc)
        adc = jnp.where(down_left, jnp.full((N_LANES,), 8, jnp.int32), adc)

        action = jnp.full((N_LANES,), 1, jnp.int32)
        action = jnp.where(cond_x_close, act_x_close, action)
        action = jnp.where(cond_x_far, act_x_far, action)
        action = jnp.where(cond_y, act_y, action)
        action = jnp.where(cond_y & cond_x_far, adf, action)
        action = jnp.where(cond_y & cond_x_close, adc, action)

        one = jnp.full((N_LANES,), 1, jnp.int32)
        return bits | lax.shift_left(one, action)

    bits = jnp.zeros((N_LANES,), jnp.int32)
    for stage in range(N_STAGES):
        pltpu.sync_copy(
            ram_hbm.at[
                pl.ds(wid * ROWS_PER + stage * STAGE_ROWS, STAGE_ROWS),
                pl.ds(COL0, 4)],
            buf)
        bits = lax.fori_loop(0, GROUPS, body, bits)

    for s in (1, 2, 4, 8):
        rot = (lanes + s) & (N_LANES - 1)
        bits = bits | bits.at[rot].get(mode="promise_in_bounds")
    ind = (lax.shift_right_logical(bits, lanes) & 1)
    ind_v[...] = ind.astype(jnp.float32)
    idx_v[...] = lanes
    z_v[...] = jnp.zeros((N_LANES,), jnp.float32)

    @pl.when(wid == 0)
    def _init():
        pltpu.sync_copy(z_v, shared)

    plsc.subcore_barrier()
    pltpu.sync_copy(ind_v, shared.at[idx_v], add=True)
    plsc.subcore_barrier()

    @pl.when(wid == 0)
    def _finalize():
        pltpu.sync_copy(shared, acc_v)
        seen = acc_v[...]
        out_v[pl.ds(0, N_LANES)] = jnp.where(
            seen > 0.0, jnp.full((N_LANES,), 1.0, jnp.float32),
            jnp.zeros((N_LANES,), jnp.float32))
        out_v[pl.ds(N_LANES, N_LANES)] = jnp.zeros((N_LANES,), jnp.float32)
        pltpu.sync_copy(out_v, out_hbm)


@jax.jit
def kernel(ram):
    mesh = plsc.VectorSubcoreMesh(
        core_axis_name="c", subcore_axis_name="s", num_cores=1)
    out32 = pl.kernel(
        _sc_body,
        out_type=jax.ShapeDtypeStruct((2 * N_LANES,), jnp.float32),
        mesh=mesh,
        compiler_params=pltpu.CompilerParams(
            use_tc_tiling_on_sc=False, needs_layout_passes=False),
        scratch_types=[
            pltpu.VMEM((STAGE_ROWS, 4), jnp.float32),
            pltpu.VMEM((N_LANES,), jnp.float32),
            pltpu.VMEM((N_LANES,), jnp.int32),
            pltpu.VMEM((N_LANES,), jnp.float32),
            pltpu.VMEM((N_LANES,), jnp.float32),
            pltpu.VMEM((2 * N_LANES,), jnp.float32),
            pltpu.VMEM_SHARED((N_LANES,), jnp.float32),
        ],
    )(ram)
    return out32[:18].reshape(1, 18)


# speedup vs baseline: 23.1925x; 1.6460x over previous
import jax
import jax.numpy as jnp
from jax import lax
from jax.experimental import pallas as pl
from jax.experimental.pallas import tpu as pltpu
from jax.experimental.pallas import tpu_sc as plsc

N_ROWS = 262144
N_CORES = 2
N_SUBCORES = 16
N_WORKERS = N_CORES * N_SUBCORES
ROWS_PER = N_ROWS // N_WORKERS
N_LANES = 16
UNROLL = 4
GROUPS = ROWS_PER // (N_LANES * UNROLL)
COL0 = 32


def _sc_body(ram_hbm, out_hbm, buf, ind_v, idx_v, z_v, acc_v, out_v, shared):
    cid = lax.axis_index("c")
    sid = lax.axis_index("s")
    wid = sid * N_CORES + cid
    lanes = lax.iota(jnp.int32, N_LANES)

    pltpu.sync_copy(
        ram_hbm.at[pl.ds(wid * ROWS_PER, ROWS_PER), pl.ds(COL0, 4)], buf)

    def one_group(rows, bits):
        c0 = jnp.zeros((N_LANES,), jnp.int32)
        mi_x = plsc.load_gather(buf, [rows, c0])
        su_x = plsc.load_gather(buf, [rows, c0 + 1])
        mi_y = plsc.load_gather(buf, [rows, c0 + 2])
        su_y = plsc.load_gather(buf, [rows, c0 + 3])

        dist_x = jnp.abs(su_x - mi_x)
        dist_y = jnp.abs(su_y - mi_y)
        go_down = su_y > mi_y
        go_up = su_y < mi_y
        go_right = su_x > mi_x
        go_left = su_x < mi_x
        cond_y = dist_y > 2.0
        cond_x_far = dist_x > 24.0
        cond_x_close = dist_x < 22.0

        def iv(v):
            return jnp.full((N_LANES,), v, jnp.int32)

        act_y = jnp.where(go_down, iv(5), iv(2))
        act_x_far = jnp.where(go_right, iv(3), iv(4))
        act_x_close = jnp.where(go_right, iv(4), iv(3))

        up_left = go_up & go_left
        down_right = go_down & go_right
        down_left = go_down & go_left

        adf = iv(6)
        adf = jnp.where(up_left, iv(7), adf)
        adf = jnp.where(down_right, iv(8), adf)
        adf = jnp.where(down_left, iv(9), adf)

        adc = iv(7)
        adc = jnp.where(up_left, iv(6), adc)
        adc = jnp.where(down_right, iv(9), adc)
        adc = jnp.where(down_left, iv(8), adc)

        action = iv(1)
        action = jnp.where(cond_x_close, act_x_close, action)
        action = jnp.where(cond_x_far, act_x_far, action)
        action = jnp.where(cond_y, act_y, action)
        action = jnp.where(cond_y & cond_x_far, adf, action)
        action = jnp.where(cond_y & cond_x_close, adc, action)

        return bits | lax.shift_left(iv(1), action)

    def body(i, bits):
        base = i * (N_LANES * UNROLL)
        for u in range(UNROLL):
            bits = one_group(base + u * N_LANES + lanes, bits)
        return bits

    bits = lax.fori_loop(0, GROUPS, body, jnp.zeros((N_LANES,), jnp.int32))

    for s in (1, 2, 4, 8):
        rot = (lanes + s) & (N_LANES - 1)
        bits = bits | bits.at[rot].get(mode="promise_in_bounds")
    ind = lax.shift_right_logical(bits, lanes) & 1
    ind_v[...] = ind.astype(jnp.float32)
    idx_v[...] = lanes
    z_v[...] = jnp.zeros((N_LANES,), jnp.float32)

    @pl.when(sid == 0)
    def _init():
        pltpu.sync_copy(z_v, shared)

    plsc.subcore_barrier()
    pltpu.sync_copy(ind_v, shared.at[idx_v], add=True)
    plsc.subcore_barrier()

    @pl.when(sid == 0)
    def _finalize():
        pltpu.sync_copy(shared, acc_v)
        seen = acc_v[...]
        out_v[pl.ds(0, N_LANES)] = jnp.where(
            seen > 0.0, jnp.full((N_LANES,), 1.0, jnp.float32),
            jnp.zeros((N_LANES,), jnp.float32))
        out_v[pl.ds(N_LANES, N_LANES)] = jnp.zeros((N_LANES,), jnp.float32)
        pltpu.sync_copy(out_v, out_hbm.at[cid])


@jax.jit
def kernel(ram):
    mesh = plsc.VectorSubcoreMesh(
        core_axis_name="c", subcore_axis_name="s", num_cores=N_CORES)
    parts = pl.kernel(
        _sc_body,
        out_type=jax.ShapeDtypeStruct((N_CORES, 2 * N_LANES), jnp.float32),
        mesh=mesh,
        compiler_params=pltpu.CompilerParams(
            use_tc_tiling_on_sc=False, needs_layout_passes=False),
        scratch_types=[
            pltpu.VMEM((ROWS_PER, 4), jnp.float32),
            pltpu.VMEM((N_LANES,), jnp.float32),
            pltpu.VMEM((N_LANES,), jnp.int32),
            pltpu.VMEM((N_LANES,), jnp.float32),
            pltpu.VMEM((N_LANES,), jnp.float32),
            pltpu.VMEM((2 * N_LANES,), jnp.float32),
            pltpu.VMEM_SHARED((N_LANES,), jnp.float32),
        ],
    )(ram)
    merged = jnp.maximum(parts[0], parts[1])
    return merged[:18].reshape(1, 18)
